# baseline (speedup 1.0000x reference)
import dataclasses

import jax
import jax.numpy as jnp
from jax import lax
from jax.experimental import pallas as pl
from jax.experimental.pallas import tpu as pltpu
from jax.experimental.pallas import tpu_sc as plsc

_NUM_CORES = 2
_NUM_SUBCORES = 16
_NUM_WORKERS = _NUM_CORES * _NUM_SUBCORES
_CHUNK = 400
_STAGE_ROWS = 2000


def _stage_linear(table):
    vocab, d = table.shape

    def body(t_ref, o_ref):
        x = t_ref[...].reshape(_STAGE_ROWS // 2, 2, d)
        o_ref[:, 0:d] = x[:, 0, :]
        o_ref[:, d : 2 * d] = x[:, 1, :]

    return pl.pallas_call(
        body,
        grid=(vocab // _STAGE_ROWS,),
        in_specs=[pl.BlockSpec((_STAGE_ROWS, d), lambda i: (i, 0))],
        out_specs=pl.BlockSpec((_STAGE_ROWS // 2, 2 * d), lambda i: (i, 0)),
        out_shape=jax.ShapeDtypeStruct((vocab // 2, 2 * d), table.dtype),
    )(table)


def kernel(ids, table):
    batch, hist = ids.shape
    vocab, d = table.shape
    num_indices = batch * hist
    per_worker = num_indices // _NUM_WORKERS

    flat = ids.reshape(num_indices)
    table_lin = _stage_linear(table).reshape(vocab, d)

    mesh = plsc.VectorSubcoreMesh(core_axis_name="c", subcore_axis_name="s")
    cp = dataclasses.replace(pltpu.CompilerParams(), use_tc_tiling_on_sc=False)

    @pl.kernel(
        out_type=jax.ShapeDtypeStruct((batch, hist, d), table.dtype),
        mesh=mesh,
        scratch_types=[
            pltpu.VMEM((_CHUNK,), jnp.int32),
            pltpu.VMEM((_CHUNK, d), table.dtype),
            pltpu.SemaphoreType.DMA,
        ],
        compiler_params=cp,
    )
    def gather_kernel(table_hbm, ids_hbm, out_hbm, idx_v, rows_v, sem):
        wid = lax.axis_index("s") * _NUM_CORES + lax.axis_index("c")
        base = wid * per_worker
        b_base = wid * (per_worker // hist)
        nb = _CHUNK // hist

        @pl.loop(0, per_worker, step=_CHUNK)
        def _(off):
            pltpu.sync_copy(ids_hbm.at[pl.ds(base + off, _CHUNK)], idx_v)
            pltpu.async_copy(table_hbm.at[idx_v], rows_v, sem).wait()
            for b in range(nb):
                pltpu.sync_copy(rows_v.at[pl.ds(b * hist, hist), :],
                                out_hbm.at[b_base + off // hist + b])

    return gather_kernel(table_lin, flat)


# speedup vs baseline: 1.3932x; 1.3932x over previous
import dataclasses

import jax
import jax.numpy as jnp
from jax import lax
from jax.experimental import pallas as pl
from jax.experimental.pallas import tpu as pltpu
from jax.experimental.pallas import tpu_sc as plsc

_NUM_CORES = 2
_NUM_SUBCORES = 16
_NUM_WORKERS = _NUM_CORES * _NUM_SUBCORES
_CHUNK = 800


def kernel(ids, table):
    batch, hist = ids.shape
    vocab, d = table.shape
    num_indices = batch * hist
    per_worker = num_indices // _NUM_WORKERS
    flat = ids.reshape(num_indices)

    mesh = plsc.VectorSubcoreMesh(core_axis_name="c", subcore_axis_name="s")
    cp = dataclasses.replace(pltpu.CompilerParams(), use_tc_tiling_on_sc=False)

    @pl.kernel(
        out_type=jax.ShapeDtypeStruct((batch, hist, d), table.dtype),
        mesh=mesh,
        scratch_types=[
            pltpu.VMEM((_CHUNK,), jnp.int32),
            pltpu.VMEM((_CHUNK, d), table.dtype),
            pltpu.SemaphoreType.DMA,
        ],
        compiler_params=cp,
    )
    def gather_kernel(table_hbm, ids_hbm, out_hbm, idx_v, rows_v, sem):
        wid = lax.axis_index("s") * _NUM_CORES + lax.axis_index("c")
        base = wid * per_worker
        b_base = wid * (per_worker // hist)
        nb = _CHUNK // hist

        @pl.loop(0, per_worker, step=_CHUNK)
        def _(off):
            pltpu.sync_copy(ids_hbm.at[pl.ds(base + off, _CHUNK)], idx_v)
            pltpu.async_copy(table_hbm.at[idx_v], rows_v, sem).wait()
            for b in range(nb):
                pltpu.sync_copy(rows_v.at[pl.ds(b * hist, hist), :],
                                out_hbm.at[b_base + off // hist + b])

    return gather_kernel(table, flat)


# speedup vs baseline: 1.4035x; 1.0074x over previous
import dataclasses

import jax
import jax.numpy as jnp
from jax import lax
from jax.experimental import pallas as pl
from jax.experimental.pallas import tpu as pltpu
from jax.experimental.pallas import tpu_sc as plsc

_NUM_CORES = 2
_NUM_SUBCORES = 16
_NUM_WORKERS = _NUM_CORES * _NUM_SUBCORES
_CHUNK = 1600


def kernel(ids, table):
    batch, hist = ids.shape
    vocab, d = table.shape
    num_indices = batch * hist
    per_worker = num_indices // _NUM_WORKERS
    flat = ids.reshape(num_indices)

    mesh = plsc.VectorSubcoreMesh(core_axis_name="c", subcore_axis_name="s")
    cp = dataclasses.replace(pltpu.CompilerParams(), use_tc_tiling_on_sc=False)

    @pl.kernel(
        out_type=jax.ShapeDtypeStruct((batch, hist, d), table.dtype),
        mesh=mesh,
        scratch_types=[
            pltpu.VMEM((_CHUNK,), jnp.int32),
            pltpu.VMEM((_CHUNK, d), table.dtype),
            pltpu.SemaphoreType.DMA,
        ],
        compiler_params=cp,
    )
    def gather_kernel(table_hbm, ids_hbm, out_hbm, idx_v, rows_v, sem):
        wid = lax.axis_index("s") * _NUM_CORES + lax.axis_index("c")
        base = wid * per_worker
        b_base = wid * (per_worker // hist)
        nb = _CHUNK // hist

        @pl.loop(0, per_worker, step=_CHUNK)
        def _(off):
            pltpu.sync_copy(ids_hbm.at[pl.ds(base + off, _CHUNK)], idx_v)
            pltpu.async_copy(table_hbm.at[idx_v], rows_v, sem).wait()
            for b in range(nb):
                pltpu.sync_copy(rows_v.at[pl.ds(b * hist, hist), :],
                                out_hbm.at[b_base + off // hist + b])

    return gather_kernel(table, flat)
